# Initial kernel scaffold; baseline (speedup 1.0000x reference)
import jax
import jax.numpy as jnp
from jax.experimental import pallas as pl


def kernel(x, Wr, br, W1, b1, W2, b2):
    raise NotImplementedError("write your pallas kernel here")


# speedup vs baseline: 1.4632x; 1.4632x over previous
import functools

import jax
import jax.numpy as jnp
from jax.experimental import pallas as pl

D_MODEL = 1024
D_FF = 2048
E = 8
TOKENS = 2048
BT = 256
N_TB = TOKENS // BT


def _moe_dense_body(x_ref, wr_ref, br_ref, w1_ref, b1_ref, w2_ref, b2_ref,
                    out_ref):
    e = pl.program_id(0)
    tb = pl.program_id(1)

    xb = x_ref[...]
    logits = jnp.dot(xb, wr_ref[...].T,
                     preferred_element_type=jnp.float32) + br_ref[...]
    lmax = jnp.max(logits, axis=-1, keepdims=True)
    p = jnp.exp(logits - lmax)
    probs = p / jnp.sum(p, axis=-1, keepdims=True)

    idx = jax.lax.broadcasted_iota(jnp.int32, probs.shape, 1)
    m1 = jnp.max(probs, axis=-1, keepdims=True)
    big = jnp.int32(E + 1)
    i1 = jnp.min(jnp.where(probs == m1, idx, big), axis=-1, keepdims=True)
    oh1 = idx == i1
    masked = jnp.where(oh1, -jnp.inf, probs)
    m2 = jnp.max(masked, axis=-1, keepdims=True)
    i2 = jnp.min(jnp.where((masked == m2) & (~oh1), idx, big),
                 axis=-1, keepdims=True)
    ge = (jnp.where(i1 == e, m1, 0.0) + jnp.where(i2 == e, m2, 0.0))
    ge = ge / (m1 + m2)

    h = jnp.maximum(
        jnp.dot(xb, w1_ref[0], preferred_element_type=jnp.float32)
        + b1_ref[0], 0.0)
    y = jnp.dot(h, w2_ref[0], preferred_element_type=jnp.float32) + b2_ref[0]
    contrib = ge * y

    sl = pl.ds(tb * BT, BT)

    @pl.when(e == 0)
    def _init():
        out_ref[sl, :] = contrib

    @pl.when(e != 0)
    def _acc():
        out_ref[sl, :] += contrib


@functools.partial(jax.jit)
def kernel(x, Wr, br, W1, b1, W2, b2):
    grid = (E, N_TB)
    return pl.pallas_call(
        _moe_dense_body,
        grid=grid,
        in_specs=[
            pl.BlockSpec((BT, D_MODEL), lambda e, tb: (tb, 0)),
            pl.BlockSpec((E, D_MODEL), lambda e, tb: (0, 0)),
            pl.BlockSpec((E,), lambda e, tb: (0,)),
            pl.BlockSpec((1, D_MODEL, D_FF), lambda e, tb: (e, 0, 0)),
            pl.BlockSpec((1, 1, D_FF), lambda e, tb: (e, 0, 0)),
            pl.BlockSpec((1, D_FF, D_MODEL), lambda e, tb: (e, 0, 0)),
            pl.BlockSpec((1, 1, D_MODEL), lambda e, tb: (e, 0, 0)),
        ],
        out_specs=pl.BlockSpec((TOKENS, D_MODEL), lambda e, tb: (0, 0)),
        out_shape=jax.ShapeDtypeStruct((TOKENS, D_MODEL), jnp.float32),
    )(x, Wr, br, W1, b1.reshape(E, 1, D_FF), W2, b2.reshape(E, 1, D_MODEL))
